# baseline (speedup 1.0000x reference)
import jax
import jax.numpy as jnp
from jax import lax
from jax.experimental import pallas as pl

HID = 128
HEADS = 4
DHEAD = 66
GDIM = 264
K_PE = 8
EPS = 1e-5
B, T, N = 2, 20, 2048
E = 65536

_HI = jax.lax.Precision.HIGHEST


def _lstm_dir(seq, Wih, Whh, bih, bhh, reverse):
    S = seq.shape[0]
    xs = jnp.swapaxes(seq, 0, 1)
    if reverse:
        xs = xs[::-1]
    h0 = jnp.zeros((S, HID), seq.dtype)

    def step(carry, xt):
        h, c = carry
        g = xt @ Wih.T + h @ Whh.T + bih + bhh
        i, f, gg, o = jnp.split(g, 4, axis=-1)
        c = jax.nn.sigmoid(f) * c + jax.nn.sigmoid(i) * jnp.tanh(gg)
        h = jax.nn.sigmoid(o) * jnp.tanh(c)
        return (h, c), h

    _, hs = jax.lax.scan(step, (h0, h0), xs)
    if reverse:
        hs = hs[::-1]
    return jnp.swapaxes(hs, 0, 1)


def _lstm(seq, p):
    out = seq
    for l in range(3):
        f = _lstm_dir(out, p['W_ih_l%d' % l], p['W_hh_l%d' % l],
                      p['b_ih_l%d' % l], p['b_hh_l%d' % l], False)
        b = _lstm_dir(out, p['W_ih_l%d_r' % l], p['W_hh_l%d_r' % l],
                      p['b_ih_l%d_r' % l], p['b_hh_l%d_r' % l], True)
        out = jnp.concatenate([f, b], axis=-1)
    return out


def _proj_body(h_ref, w_ref, b_ref, out_ref):
    out_ref[...] = (
        lax.dot_general(h_ref[...], w_ref[...], (((1,), (0,)), ((), ())),
                        precision=_HI, preferred_element_type=jnp.float32)
        + b_ref[...]
    )


def _attn_body(q_ref, k_ref, v_ref, ct_ref, out_ref):
    ct = ct_ref[...]
    mask = ct > 0.0
    neg = jnp.float32(-1e30)
    inv_sqrt_d = jnp.float32(1.0) / jnp.sqrt(jnp.float32(DHEAD))
    for hh in range(HEADS):
        q = q_ref[0, hh]
        k = k_ref[0, hh]
        v = v_ref[0, hh]
        alpha = lax.dot_general(q, k, (((1,), (1,)), ((), ())),
                                precision=_HI,
                                preferred_element_type=jnp.float32)
        alpha = alpha * inv_sqrt_d
        am = jnp.where(mask, alpha, neg)
        m = jnp.max(am, axis=1, keepdims=True)
        m = jnp.where(m > jnp.float32(-1e29), m, 0.0)
        w = ct * jnp.exp(am - m)
        s = jnp.sum(w, axis=1, keepdims=True)
        a = w / (s + jnp.float32(1e-16))
        out_ref[0, hh] = lax.dot_general(a, v, (((1,), (0,)), ((), ())),
                                         precision=_HI,
                                         preferred_element_type=jnp.float32)


def _post_body(agg_ref, sk_ref, hin_ref, w_ref, b_ref, ms_ref, out_ref):
    xa = agg_ref[...] + sk_ref[...]
    n = jnp.float32(xa.shape[0])
    mean = jnp.sum(xa, axis=0, keepdims=True) / n
    cen = xa - ms_ref[...] * mean
    var = jnp.sum(cen * cen, axis=0, keepdims=True) / n
    gn = cen / jnp.sqrt(var + jnp.float32(EPS)) * w_ref[...] + b_ref[...]
    out_ref[...] = jnp.maximum(gn + hin_ref[...], 0.0)


def _pool_cls_body(h_ref, w_ref, b_ref, out_ref):
    p0 = jnp.sum(h_ref[0:N, :], axis=0, keepdims=True) / jnp.float32(N)
    p1 = jnp.sum(h_ref[N:2 * N, :], axis=0, keepdims=True) / jnp.float32(N)
    pooled = jnp.concatenate([p0, p1], axis=0)
    out_ref[...] = (
        lax.dot_general(pooled, w_ref[...], (((1,), (0,)), ((), ())),
                        precision=_HI, preferred_element_type=jnp.float32)
        + b_ref[...]
    )


_BD = 256
_PROJ_ROWS = 512


def _tconv_dense(h, Ct, p, i):
    Nn = B * N
    wcat = jnp.concatenate(
        [p['tf%d_%s' % (i, nm)].T for nm in ('Wq', 'Wk', 'Wv', 'Ws')], axis=1)
    bcat = jnp.concatenate(
        [p['tf%d_%s' % (i, nm)] for nm in ('bq', 'bk', 'bv', 'bs')])[None, :]

    qkvs = pl.pallas_call(
        _proj_body,
        grid=(Nn // _PROJ_ROWS,),
        in_specs=[
            pl.BlockSpec((_PROJ_ROWS, GDIM), lambda r: (r, 0)),
            pl.BlockSpec((GDIM, 4 * GDIM), lambda r: (0, 0)),
            pl.BlockSpec((1, 4 * GDIM), lambda r: (0, 0)),
        ],
        out_specs=pl.BlockSpec((_PROJ_ROWS, 4 * GDIM), lambda r: (r, 0)),
        out_shape=jax.ShapeDtypeStruct((Nn, 4 * GDIM), jnp.float32),
    )(h, wcat, bcat)

    def heads(c):
        return (qkvs[:, c * GDIM:(c + 1) * GDIM]
                .reshape(B, N, HEADS, DHEAD).transpose(0, 2, 1, 3))

    q, k, v = heads(0), heads(1), heads(2)
    sk = qkvs[:, 3 * GDIM:4 * GDIM]

    agg = pl.pallas_call(
        _attn_body,
        grid=(B, N // _BD),
        in_specs=[
            pl.BlockSpec((1, HEADS, _BD, DHEAD), lambda g, r: (g, 0, r, 0)),
            pl.BlockSpec((1, HEADS, N, DHEAD), lambda g, r: (g, 0, 0, 0)),
            pl.BlockSpec((1, HEADS, N, DHEAD), lambda g, r: (g, 0, 0, 0)),
            pl.BlockSpec((_BD, N), lambda g, r: (r, 0)),
        ],
        out_specs=pl.BlockSpec((1, HEADS, _BD, DHEAD), lambda g, r: (g, 0, r, 0)),
        out_shape=jax.ShapeDtypeStruct((B, HEADS, N, DHEAD), jnp.float32),
    )(q, k, v, Ct)

    agg_flat = agg.transpose(0, 2, 1, 3).reshape(Nn, GDIM)

    return pl.pallas_call(
        _post_body,
        grid=(B,),
        in_specs=[
            pl.BlockSpec((N, GDIM), lambda g: (g, 0)),
            pl.BlockSpec((N, GDIM), lambda g: (g, 0)),
            pl.BlockSpec((N, GDIM), lambda g: (g, 0)),
            pl.BlockSpec((1, GDIM), lambda g: (0, 0)),
            pl.BlockSpec((1, GDIM), lambda g: (0, 0)),
            pl.BlockSpec((1, GDIM), lambda g: (0, 0)),
        ],
        out_specs=pl.BlockSpec((N, GDIM), lambda g: (g, 0)),
        out_shape=jax.ShapeDtypeStruct((Nn, GDIM), jnp.float32),
    )(agg_flat, sk, h,
      p['gn%d_w' % i][None, :], p['gn%d_b' % i][None, :],
      p['gn%d_ms' % i][None, :])


def kernel(x, edge_index, params):
    p = params
    src0, dst0 = edge_index[0], edge_index[1]

    A_raw = jnp.zeros((N, N), jnp.float32).at[src0, dst0].add(1.0)
    A = 0.5 * (A_raw + A_raw.T)
    d = A.sum(axis=1)
    dinv = jnp.where(d > 0, 1.0 / jnp.sqrt(jnp.maximum(d, 1e-12)), 0.0)
    L = jnp.eye(N, dtype=jnp.float32) - dinv[:, None] * A * dinv[None, :]
    _, v = jnp.linalg.eigh(L)
    pe = jnp.tile(v[:, 1:K_PE + 1], (B, 1))

    src2, dst2 = jax.lax.optimization_barrier((src0, dst0))
    C_raw = jnp.zeros((N, N), jnp.float32).at[src2, dst2].add(1.0)
    Ct = C_raw.T

    seq = jnp.transpose(x, (0, 2, 1)).reshape(B * N, T, 1)
    lo = _lstm(seq, p)
    feats = lo.mean(axis=1)
    h = jnp.concatenate([feats, pe], axis=-1)

    for i in (1, 2, 3):
        h = _tconv_dense(h, Ct, p, i)

    w_pad = jnp.zeros((GDIM, 128), jnp.float32).at[:, 0].set(p['cls_W'][0])
    b_pad = jnp.zeros((1, 128), jnp.float32).at[0, 0].set(p['cls_b'][0])
    out = pl.pallas_call(
        _pool_cls_body,
        out_shape=jax.ShapeDtypeStruct((B, 128), jnp.float32),
    )(h, w_pad, b_pad)
    return out[:, :1]


# speedup vs baseline: 1.0019x; 1.0019x over previous
import jax
import jax.numpy as jnp
from jax import lax
from jax.experimental import pallas as pl
from jax.experimental.pallas import tpu as pltpu
from jax.experimental.pallas import tpu_sc as plsc

HID = 128
HEADS = 4
DHEAD = 66
GDIM = 264
K_PE = 8
EPS = 1e-5
B, T, N = 2, 20, 2048
E = 65536

_HI = jax.lax.Precision.HIGHEST


def _lstm_dir(seq, Wih, Whh, bih, bhh, reverse):
    S = seq.shape[0]
    xs = jnp.swapaxes(seq, 0, 1)
    if reverse:
        xs = xs[::-1]
    h0 = jnp.zeros((S, HID), seq.dtype)

    def step(carry, xt):
        h, c = carry
        g = xt @ Wih.T + h @ Whh.T + bih + bhh
        i, f, gg, o = jnp.split(g, 4, axis=-1)
        c = jax.nn.sigmoid(f) * c + jax.nn.sigmoid(i) * jnp.tanh(gg)
        h = jax.nn.sigmoid(o) * jnp.tanh(c)
        return (h, c), h

    _, hs = jax.lax.scan(step, (h0, h0), xs)
    if reverse:
        hs = hs[::-1]
    return jnp.swapaxes(hs, 0, 1)


def _lstm(seq, p):
    out = seq
    for l in range(3):
        f = _lstm_dir(out, p['W_ih_l%d' % l], p['W_hh_l%d' % l],
                      p['b_ih_l%d' % l], p['b_hh_l%d' % l], False)
        b = _lstm_dir(out, p['W_ih_l%d_r' % l], p['W_hh_l%d_r' % l],
                      p['b_ih_l%d_r' % l], p['b_hh_l%d_r' % l], True)
        out = jnp.concatenate([f, b], axis=-1)
    return out


_SC_ROWS = 32
_EBLK = 2048


def _count_body(dst_hbm, src_hbm, out_hbm, dvm, svm, buf):
    wid = lax.axis_index("s") * 2 + lax.axis_index("c")
    for chunk in range(2):
        base = (wid * 2 + chunk) * _SC_ROWS

        def zrow(r, carry):
            def zcol(j, c2):
                buf[r, pl.ds(j * 16, 16)] = jnp.zeros((16,), jnp.float32)
                return c2
            return lax.fori_loop(0, N // 16, zcol, carry)
        lax.fori_loop(0, _SC_ROWS, zrow, 0)

        def blk_body(bi, carry):
            pltpu.sync_copy(dst_hbm.at[pl.ds(bi * _EBLK, _EBLK)], dvm)
            pltpu.sync_copy(src_hbm.at[pl.ds(bi * _EBLK, _EBLK)], svm)

            def vec_body(j, c2):
                d = dvm[pl.ds(j * 16, 16)]
                s = svm[pl.ds(j * 16, 16)]
                ld = d - base
                ok = (ld >= 0) & (ld < _SC_ROWS)
                lds = jnp.where(ok, ld, 0)
                plsc.addupdate_scatter(buf, [lds, s],
                                       jnp.ones((16,), jnp.float32),
                                       mask=ok)
                return c2
            return lax.fori_loop(0, _EBLK // 16, vec_body, carry)
        lax.fori_loop(0, E // _EBLK, blk_body, 0)

        pltpu.sync_copy(buf, out_hbm.at[pl.ds(base, _SC_ROWS)])


def _count_matrix(dst0, src0):
    return pl.kernel(
        _count_body,
        out_type=jax.ShapeDtypeStruct((N, N), jnp.float32),
        mesh=plsc.VectorSubcoreMesh(core_axis_name="c", subcore_axis_name="s"),
        compiler_params=pltpu.CompilerParams(needs_layout_passes=False),
        scratch_types=[
            pltpu.VMEM((_EBLK,), jnp.int32),
            pltpu.VMEM((_EBLK,), jnp.int32),
            pltpu.VMEM((_SC_ROWS, N), jnp.float32),
        ],
    )(dst0, src0)


def _proj_body(h_ref, w_ref, b_ref, out_ref):
    out_ref[...] = (
        lax.dot_general(h_ref[...], w_ref[...], (((1,), (0,)), ((), ())),
                        precision=_HI, preferred_element_type=jnp.float32)
        + b_ref[...]
    )


def _attn_body(q_ref, k_ref, v_ref, ct_ref, out_ref):
    ct = ct_ref[...]
    mask = ct > 0.0
    neg = jnp.float32(-1e30)
    inv_sqrt_d = jnp.float32(1.0) / jnp.sqrt(jnp.float32(DHEAD))
    for hh in range(HEADS):
        q = q_ref[0, hh]
        k = k_ref[0, hh]
        v = v_ref[0, hh]
        alpha = lax.dot_general(q, k, (((1,), (1,)), ((), ())),
                                precision=_HI,
                                preferred_element_type=jnp.float32)
        alpha = alpha * inv_sqrt_d
        am = jnp.where(mask, alpha, neg)
        m = jnp.max(am, axis=1, keepdims=True)
        m = jnp.where(m > jnp.float32(-1e29), m, 0.0)
        w = ct * jnp.exp(am - m)
        s = jnp.sum(w, axis=1, keepdims=True)
        a = w / (s + jnp.float32(1e-16))
        out_ref[0, hh] = lax.dot_general(a, v, (((1,), (0,)), ((), ())),
                                         precision=_HI,
                                         preferred_element_type=jnp.float32)


def _post_body(agg_ref, sk_ref, hin_ref, w_ref, b_ref, ms_ref, out_ref):
    xa = agg_ref[...] + sk_ref[...]
    n = jnp.float32(xa.shape[0])
    mean = jnp.sum(xa, axis=0, keepdims=True) / n
    cen = xa - ms_ref[...] * mean
    var = jnp.sum(cen * cen, axis=0, keepdims=True) / n
    gn = cen / jnp.sqrt(var + jnp.float32(EPS)) * w_ref[...] + b_ref[...]
    out_ref[...] = jnp.maximum(gn + hin_ref[...], 0.0)


def _pool_cls_body(h_ref, w_ref, b_ref, out_ref):
    p0 = jnp.sum(h_ref[0:N, :], axis=0, keepdims=True) / jnp.float32(N)
    p1 = jnp.sum(h_ref[N:2 * N, :], axis=0, keepdims=True) / jnp.float32(N)
    pooled = jnp.concatenate([p0, p1], axis=0)
    out_ref[...] = (
        lax.dot_general(pooled, w_ref[...], (((1,), (0,)), ((), ())),
                        precision=_HI, preferred_element_type=jnp.float32)
        + b_ref[...]
    )


_BD = 256
_PROJ_ROWS = 512


def _tconv_dense(h, Ct, p, i):
    Nn = B * N
    wcat = jnp.concatenate(
        [p['tf%d_%s' % (i, nm)].T for nm in ('Wq', 'Wk', 'Wv', 'Ws')], axis=1)
    bcat = jnp.concatenate(
        [p['tf%d_%s' % (i, nm)] for nm in ('bq', 'bk', 'bv', 'bs')])[None, :]

    qkvs = pl.pallas_call(
        _proj_body,
        grid=(Nn // _PROJ_ROWS,),
        in_specs=[
            pl.BlockSpec((_PROJ_ROWS, GDIM), lambda r: (r, 0)),
            pl.BlockSpec((GDIM, 4 * GDIM), lambda r: (0, 0)),
            pl.BlockSpec((1, 4 * GDIM), lambda r: (0, 0)),
        ],
        out_specs=pl.BlockSpec((_PROJ_ROWS, 4 * GDIM), lambda r: (r, 0)),
        out_shape=jax.ShapeDtypeStruct((Nn, 4 * GDIM), jnp.float32),
    )(h, wcat, bcat)

    def heads(c):
        return (qkvs[:, c * GDIM:(c + 1) * GDIM]
                .reshape(B, N, HEADS, DHEAD).transpose(0, 2, 1, 3))

    q, k, v = heads(0), heads(1), heads(2)
    sk = qkvs[:, 3 * GDIM:4 * GDIM]

    agg = pl.pallas_call(
        _attn_body,
        grid=(B, N // _BD),
        in_specs=[
            pl.BlockSpec((1, HEADS, _BD, DHEAD), lambda g, r: (g, 0, r, 0)),
            pl.BlockSpec((1, HEADS, N, DHEAD), lambda g, r: (g, 0, 0, 0)),
            pl.BlockSpec((1, HEADS, N, DHEAD), lambda g, r: (g, 0, 0, 0)),
            pl.BlockSpec((_BD, N), lambda g, r: (r, 0)),
        ],
        out_specs=pl.BlockSpec((1, HEADS, _BD, DHEAD), lambda g, r: (g, 0, r, 0)),
        out_shape=jax.ShapeDtypeStruct((B, HEADS, N, DHEAD), jnp.float32),
    )(q, k, v, Ct)

    agg_flat = agg.transpose(0, 2, 1, 3).reshape(Nn, GDIM)

    return pl.pallas_call(
        _post_body,
        grid=(B,),
        in_specs=[
            pl.BlockSpec((N, GDIM), lambda g: (g, 0)),
            pl.BlockSpec((N, GDIM), lambda g: (g, 0)),
            pl.BlockSpec((N, GDIM), lambda g: (g, 0)),
            pl.BlockSpec((1, GDIM), lambda g: (0, 0)),
            pl.BlockSpec((1, GDIM), lambda g: (0, 0)),
            pl.BlockSpec((1, GDIM), lambda g: (0, 0)),
        ],
        out_specs=pl.BlockSpec((N, GDIM), lambda g: (g, 0)),
        out_shape=jax.ShapeDtypeStruct((Nn, GDIM), jnp.float32),
    )(agg_flat, sk, h,
      p['gn%d_w' % i][None, :], p['gn%d_b' % i][None, :],
      p['gn%d_ms' % i][None, :])


def kernel(x, edge_index, params):
    p = params
    src0, dst0 = edge_index[0], edge_index[1]

    Ct = _count_matrix(dst0, src0)
    A_raw = Ct.T
    A = 0.5 * (A_raw + A_raw.T)
    d = A.sum(axis=1)
    dinv = jnp.where(d > 0, 1.0 / jnp.sqrt(jnp.maximum(d, 1e-12)), 0.0)
    L = jnp.eye(N, dtype=jnp.float32) - dinv[:, None] * A * dinv[None, :]
    _, v = jnp.linalg.eigh(L)
    pe = jnp.tile(v[:, 1:K_PE + 1], (B, 1))

    seq = jnp.transpose(x, (0, 2, 1)).reshape(B * N, T, 1)
    lo = _lstm(seq, p)
    feats = lo.mean(axis=1)
    h = jnp.concatenate([feats, pe], axis=-1)

    for i in (1, 2, 3):
        h = _tconv_dense(h, Ct, p, i)

    w_pad = jnp.zeros((GDIM, 128), jnp.float32).at[:, 0].set(p['cls_W'][0])
    b_pad = jnp.zeros((1, 128), jnp.float32).at[0, 0].set(p['cls_b'][0])
    out = pl.pallas_call(
        _pool_cls_body,
        out_shape=jax.ShapeDtypeStruct((B, 128), jnp.float32),
    )(h, w_pad, b_pad)
    return out[:, :1]
